# baseline (speedup 1.0000x reference)
import functools

import jax
import jax.numpy as jnp
from jax import lax
from jax.experimental import pallas as pl
from jax.experimental.pallas import tpu as pltpu
from jax.experimental.pallas import tpu_sc as plsc

_RES = (16, 32, 64, 128)
_NC = 2
_NS = 16
_NW = _NC * _NS
_CG = 128
_GROUPS = _CG // 16


def _sc_interp(coords, g0, g1, g2, g3, n_points):
  pw = n_points // _NW
  nchunk = pw // _CG

  mesh = plsc.VectorSubcoreMesh(
      core_axis_name="c", subcore_axis_name="s",
      num_cores=_NC, num_subcores=_NS)

  @functools.partial(
      pl.kernel,
      out_type=jax.ShapeDtypeStruct((n_points, 32), jnp.float32),
      mesh=mesh,
      compiler_params=pltpu.CompilerParams(
          needs_layout_passes=False, use_tc_tiling_on_sc=False),
      scratch_types=[
          pltpu.VMEM((2 * _CG, 3), jnp.float32),
          pltpu.VMEM((64, _CG), jnp.int32),
          pltpu.VMEM((24, _CG), jnp.float32),
          pltpu.VMEM((64 * _CG, 8), jnp.float32),
          pltpu.VMEM((2 * _CG, 32), jnp.float32),
          pltpu.SemaphoreType.DMA((2,)),
      ],
  )
  def body(coords_hbm, tab0, tab1, tab2, tab3, out_hbm,
           cbuf, idxb, wbuf, rows, outb, sems):
    tables = (tab0, tab1, tab2, tab3)
    wid = lax.axis_index("s") * _NC + lax.axis_index("c")
    iota = lax.iota(jnp.int32, 16)

    def prep(k, h):
      base = wid * pw + k * _CG
      pltpu.sync_copy(coords_hbm.at[pl.ds(base, _CG)],
                      cbuf.at[pl.ds(h * _CG, _CG)])

      @plsc.parallel_loop(0, _GROUPS, unroll=2)
      def idx_g(g):
        sl = pl.ds(g * 16, 16)
        crow = iota + (h * _CG + g * 16)
        xv = plsc.load_gather(cbuf, [crow, jnp.full((16,), 0, jnp.int32)])
        yv = plsc.load_gather(cbuf, [crow, jnp.full((16,), 1, jnp.int32)])
        zv = plsc.load_gather(cbuf, [crow, jnp.full((16,), 2, jnp.int32)])
        for l, res in enumerate(_RES):
          rm1 = float(res - 1)
          gx = jnp.clip(xv * rm1, 0.0, rm1)
          gy = jnp.clip(yv * rm1, 0.0, rm1)
          gz = jnp.clip(zv * rm1, 0.0, rm1)
          fx = gx.astype(jnp.int32)
          fy = gy.astype(jnp.int32)
          fz = gz.astype(jnp.int32)
          wbuf[12 * h + l * 3 + 0, sl] = gx - fx.astype(jnp.float32)
          wbuf[12 * h + l * 3 + 1, sl] = gy - fy.astype(jnp.float32)
          wbuf[12 * h + l * 3 + 2, sl] = gz - fz.astype(jnp.float32)
          x1 = jnp.minimum(fx + 1, res - 1)
          y1 = jnp.minimum(fy + 1, res - 1)
          z1 = jnp.minimum(fz + 1, res - 1)
          b00 = (fx * res + fy) * res
          b01 = (fx * res + y1) * res
          b10 = (x1 * res + fy) * res
          b11 = (x1 * res + y1) * res
          o = 32 * h + l * 8
          idxb[o + 0, sl] = b00 + fz
          idxb[o + 1, sl] = b00 + z1
          idxb[o + 2, sl] = b01 + fz
          idxb[o + 3, sl] = b01 + z1
          idxb[o + 4, sl] = b10 + fz
          idxb[o + 5, sl] = b10 + z1
          idxb[o + 6, sl] = b11 + fz
          idxb[o + 7, sl] = b11 + z1

      for l in range(4):
        for cc in range(8):
          i = 32 * h + l * 8 + cc
          pltpu.async_copy(tables[l].at[idxb.at[i]],
                           rows.at[pl.ds(i * _CG, _CG)], sems.at[h])

    def drain(h):
      pltpu.make_async_copy(tab3.at[pl.ds(0, 32 * _CG)],
                            rows.at[pl.ds(32 * h * _CG, 32 * _CG)],
                            sems.at[h]).wait()

    def interp(k, h):
      base = wid * pw + k * _CG

      @plsc.parallel_loop(0, _GROUPS, unroll=2)
      def interp_g(g):
        rb = iota + g * 16
        sl = pl.ds(g * 16, 16)
        for l in range(4):
          wx = wbuf[12 * h + l * 3 + 0, sl]
          wy = wbuf[12 * h + l * 3 + 1, sl]
          wz = wbuf[12 * h + l * 3 + 2, sl]
          omx = 1.0 - wx
          omy = 1.0 - wy
          p0 = omx * omy
          p1 = omx * wy
          p2 = wx * omy
          p3 = wx * wy
          o = 32 * h + l * 8
          rbs = [rb + (o + cc) * _CG for cc in range(8)]
          orow = rb + h * _CG
          for f in range(8):
            col = jnp.full((16,), f, jnp.int32)
            a0 = plsc.load_gather(rows, [rbs[0], col])
            b0 = plsc.load_gather(rows, [rbs[1], col])
            a1 = plsc.load_gather(rows, [rbs[2], col])
            b1 = plsc.load_gather(rows, [rbs[3], col])
            a2 = plsc.load_gather(rows, [rbs[4], col])
            b2 = plsc.load_gather(rows, [rbs[5], col])
            a3 = plsc.load_gather(rows, [rbs[6], col])
            b3 = plsc.load_gather(rows, [rbs[7], col])
            s0 = (a0 * p0 + a1 * p1) + (a2 * p2 + a3 * p3)
            s1 = (b0 * p0 + b1 * p1) + (b2 * p2 + b3 * p3)
            ov = s0 + (s1 - s0) * wz
            plsc.store_scatter(
                outb, [orow, jnp.full((16,), l * 8 + f, jnp.int32)], ov)

      pltpu.sync_copy(outb.at[pl.ds(h * _CG, _CG)],
                      out_hbm.at[pl.ds(base, _CG)])

    prep(0, 0)

    def chunk_body(k, carry):
      h = lax.rem(k, 2)
      hn = lax.rem(k + 1, 2)

      @pl.when(k + 1 < nchunk)
      def _():
        prep(k + 1, hn)

      drain(h)
      interp(k, h)
      return carry

    lax.fori_loop(0, nchunk, chunk_body, 0)

  return body(coords, g0, g1, g2, g3)


def _mlp_body(x_ref, w0, b0, w1, b1, w2, b2, w3, b3, o_ref):
  x = x_ref[...]
  h = jnp.dot(x, w0[...], preferred_element_type=jnp.float32) + b0[...]
  h = jnp.maximum(h, 0.0)
  h = jnp.dot(h, w1[...], preferred_element_type=jnp.float32) + b1[...]
  h = jnp.maximum(h, 0.0)
  h = jnp.dot(h, w2[...], preferred_element_type=jnp.float32) + b2[...]
  h = jnp.maximum(h, 0.0)
  o_ref[...] = jnp.dot(h, w3[...], preferred_element_type=jnp.float32) + b3[...]


def _tc_mlp(feats, W0, b0, W1, b1, W2, b2, W3, b3):
  n = feats.shape[0]
  bn = 4096
  full = lambda shape: pl.BlockSpec(shape, lambda i: (0,) * len(shape))
  return pl.pallas_call(
      _mlp_body,
      grid=(n // bn,),
      in_specs=[
          pl.BlockSpec((bn, 32), lambda i: (i, 0)),
          full(W0.shape), full((1, 64)),
          full(W1.shape), full((1, 64)),
          full(W2.shape), full((1, 64)),
          full(W3.shape), full((1, 4)),
      ],
      out_specs=pl.BlockSpec((bn, 4), lambda i: (i, 0)),
      out_shape=jax.ShapeDtypeStruct((n, 4), jnp.float32),
  )(feats, W0, b0.reshape(1, 64), W1, b1.reshape(1, 64),
    W2, b2.reshape(1, 64), W3, b3.reshape(1, 4))


def kernel(coords, grid0, grid1, grid2, grid3, W0, b0, W1, b1, W2, b2, W3, b3):
  n = coords.shape[0]
  feats = _sc_interp(coords,
                     grid0.reshape(-1, 8), grid1.reshape(-1, 8),
                     grid2.reshape(-1, 8), grid3.reshape(-1, 8), n)
  return _tc_mlp(feats, W0, b0, W1, b1, W2, b2, W3, b3)


# speedup vs baseline: 1.0861x; 1.0861x over previous
import functools

import jax
import jax.numpy as jnp
from jax import lax
from jax.experimental import pallas as pl
from jax.experimental.pallas import tpu as pltpu
from jax.experimental.pallas import tpu_sc as plsc

_RES = (16, 32, 64, 128)
_NC = 2
_NS = 16
_NW = _NC * _NS
_CG = 128
_GROUPS = _CG // 16


def _sc_interp(coords_t, t0, t1, t2, t3, n_points):
  pw = n_points // _NW
  nchunk = pw // _CG

  mesh = plsc.VectorSubcoreMesh(
      core_axis_name="c", subcore_axis_name="s",
      num_cores=_NC, num_subcores=_NS)

  @functools.partial(
      pl.kernel,
      out_type=jax.ShapeDtypeStruct((n_points, 32), jnp.float32),
      mesh=mesh,
      compiler_params=pltpu.CompilerParams(
          needs_layout_passes=False, use_tc_tiling_on_sc=False),
      scratch_types=[
          pltpu.VMEM((8, _CG), jnp.float32),
          pltpu.VMEM((32, _CG), jnp.int32),
          pltpu.VMEM((24, _CG), jnp.float32),
          pltpu.VMEM((32 * _CG, 16), jnp.float32),
          pltpu.VMEM((2 * _CG, 32), jnp.float32),
          pltpu.SemaphoreType.DMA((2,)),
      ],
  )
  def body(coords_hbm, tab0, tab1, tab2, tab3, out_hbm,
           cbuf, idxb, wbuf, rows, outb, sems):
    tables = (tab0, tab1, tab2, tab3)
    wid = lax.axis_index("s") * _NC + lax.axis_index("c")
    iota = lax.iota(jnp.int32, 16)

    def gather_refs(h, l, c):
      i = l * 4 + c
      return (tables[l].at[idxb.at[16 * h + i]],
              rows.at[pl.ds((16 * h + i) * _CG, _CG)])

    def prep(k, h):
      base = wid * pw + k * _CG
      pltpu.sync_copy(coords_hbm.at[:, pl.ds(base, _CG)],
                      cbuf.at[pl.ds(4 * h, 3)])

      @plsc.parallel_loop(0, _GROUPS, unroll=2)
      def idx_g(g):
        sl = pl.ds(g * 16, 16)
        xv = cbuf[4 * h + 0, sl]
        yv = cbuf[4 * h + 1, sl]
        zv = cbuf[4 * h + 2, sl]
        for l, res in enumerate(_RES):
          rm1 = float(res - 1)
          gx = jnp.clip(xv * rm1, 0.0, rm1)
          gy = jnp.clip(yv * rm1, 0.0, rm1)
          gz = jnp.clip(zv * rm1, 0.0, rm1)
          fx = gx.astype(jnp.int32)
          fy = gy.astype(jnp.int32)
          fz = gz.astype(jnp.int32)
          wbuf[12 * h + l * 3 + 0, sl] = gx - fx.astype(jnp.float32)
          wbuf[12 * h + l * 3 + 1, sl] = gy - fy.astype(jnp.float32)
          wbuf[12 * h + l * 3 + 2, sl] = gz - fz.astype(jnp.float32)
          x1 = jnp.minimum(fx + 1, res - 1)
          y1 = jnp.minimum(fy + 1, res - 1)
          idxb[16 * h + l * 4 + 0, sl] = (fx * res + fy) * res + fz
          idxb[16 * h + l * 4 + 1, sl] = (fx * res + y1) * res + fz
          idxb[16 * h + l * 4 + 2, sl] = (x1 * res + fy) * res + fz
          idxb[16 * h + l * 4 + 3, sl] = (x1 * res + y1) * res + fz

      for l in range(4):
        for c in range(4):
          src, dst = gather_refs(h, l, c)
          pltpu.async_copy(src, dst, sems.at[h])

    def drain(h):
      pltpu.make_async_copy(tab3.at[pl.ds(0, 16 * _CG)],
                            rows.at[pl.ds(16 * h * _CG, 16 * _CG)],
                            sems.at[h]).wait()

    def interp(k, h):
      base = wid * pw + k * _CG

      @plsc.parallel_loop(0, _GROUPS, unroll=4)
      def interp_g(g):
        rb = iota + g * 16
        sl = pl.ds(g * 16, 16)
        for l in range(4):
          wx = wbuf[12 * h + l * 3 + 0, sl]
          wy = wbuf[12 * h + l * 3 + 1, sl]
          wz = wbuf[12 * h + l * 3 + 2, sl]
          omx = 1.0 - wx
          omy = 1.0 - wy
          p0 = omx * omy
          p1 = omx * wy
          p2 = wx * omy
          p3 = wx * wy
          rbs = [rb + (16 * h + l * 4 + c) * _CG for c in range(4)]
          orow = rb + h * _CG
          for f in range(8):
            col0 = jnp.full((16,), f, jnp.int32)
            col1 = jnp.full((16,), 8 + f, jnp.int32)
            a0 = plsc.load_gather(rows, [rbs[0], col0])
            b0 = plsc.load_gather(rows, [rbs[0], col1])
            a1 = plsc.load_gather(rows, [rbs[1], col0])
            b1 = plsc.load_gather(rows, [rbs[1], col1])
            a2 = plsc.load_gather(rows, [rbs[2], col0])
            b2 = plsc.load_gather(rows, [rbs[2], col1])
            a3 = plsc.load_gather(rows, [rbs[3], col0])
            b3 = plsc.load_gather(rows, [rbs[3], col1])
            s0 = (a0 * p0 + a1 * p1) + (a2 * p2 + a3 * p3)
            s1 = (b0 * p0 + b1 * p1) + (b2 * p2 + b3 * p3)
            o = s0 + (s1 - s0) * wz
            plsc.store_scatter(
                outb, [orow, jnp.full((16,), l * 8 + f, jnp.int32)], o)

      pltpu.sync_copy(outb.at[pl.ds(h * _CG, _CG)],
                      out_hbm.at[pl.ds(base, _CG)])

    prep(0, 0)

    def chunk_body(k, carry):
      h = lax.rem(k, 2)
      hn = lax.rem(k + 1, 2)

      @pl.when(k + 1 < nchunk)
      def _():
        prep(k + 1, hn)

      drain(h)
      interp(k, h)
      return carry

    lax.fori_loop(0, nchunk, chunk_body, 0)

  return body(coords_t, t0, t1, t2, t3)


def _mlp_body(x_ref, w0, b0, w1, b1, w2, b2, w3, b3, o_ref):
  x = x_ref[...]
  h = jnp.dot(x, w0[...], preferred_element_type=jnp.float32) + b0[...]
  h = jnp.maximum(h, 0.0)
  h = jnp.dot(h, w1[...], preferred_element_type=jnp.float32) + b1[...]
  h = jnp.maximum(h, 0.0)
  h = jnp.dot(h, w2[...], preferred_element_type=jnp.float32) + b2[...]
  h = jnp.maximum(h, 0.0)
  o_ref[...] = jnp.dot(h, w3[...], preferred_element_type=jnp.float32) + b3[...]


def _tc_mlp(feats, W0, b0, W1, b1, W2, b2, W3, b3):
  n = feats.shape[0]
  bn = 4096
  full = lambda shape: pl.BlockSpec(shape, lambda i: (0,) * len(shape))
  return pl.pallas_call(
      _mlp_body,
      grid=(n // bn,),
      in_specs=[
          pl.BlockSpec((bn, 32), lambda i: (i, 0)),
          full(W0.shape), full((1, 64)),
          full(W1.shape), full((1, 64)),
          full(W2.shape), full((1, 64)),
          full(W3.shape), full((1, 4)),
      ],
      out_specs=pl.BlockSpec((bn, 4), lambda i: (i, 0)),
      out_shape=jax.ShapeDtypeStruct((n, 4), jnp.float32),
  )(feats, W0, b0.reshape(1, 64), W1, b1.reshape(1, 64),
    W2, b2.reshape(1, 64), W3, b3.reshape(1, 4))


def _pair_z(g):
  res = g.shape[0]
  gz1 = jnp.concatenate([g[:, :, 1:, :], g[:, :, -1:, :]], axis=2)
  return jnp.concatenate([g, gz1], axis=-1).reshape(res * res * res, 16)


def kernel(coords, grid0, grid1, grid2, grid3, W0, b0, W1, b1, W2, b2, W3, b3):
  n = coords.shape[0]
  coords_t = coords.T
  t0 = _pair_z(grid0)
  t1 = _pair_z(grid1)
  t2 = _pair_z(grid2)
  t3 = _pair_z(grid3)
  feats = _sc_interp(coords_t, t0, t1, t2, t3, n)
  return _tc_mlp(feats, W0, b0, W1, b1, W2, b2, W3, b3)


# speedup vs baseline: 1.1580x; 1.0662x over previous
import functools

import jax
import jax.numpy as jnp
from jax import lax
from jax.experimental import pallas as pl
from jax.experimental.pallas import tpu as pltpu
from jax.experimental.pallas import tpu_sc as plsc

_RES = (16, 32, 64, 128)
_NC = 2
_NS = 16
_NW = _NC * _NS
_CG = 128
_GROUPS = _CG // 16


def _sc_interp(coords_t, t0, t1, t2, t3, n_points):
  pw = n_points // _NW
  nchunk = pw // _CG

  mesh = plsc.VectorSubcoreMesh(
      core_axis_name="c", subcore_axis_name="s",
      num_cores=_NC, num_subcores=_NS)

  @functools.partial(
      pl.kernel,
      out_type=jax.ShapeDtypeStruct((n_points, 32), jnp.float32),
      mesh=mesh,
      compiler_params=pltpu.CompilerParams(
          needs_layout_passes=False, use_tc_tiling_on_sc=False),
      scratch_types=[
          pltpu.VMEM((8, _CG), jnp.float32),
          pltpu.VMEM((32, _CG), jnp.int32),
          pltpu.VMEM((24, _CG), jnp.float32),
          pltpu.VMEM((32 * _CG, 16), jnp.float32),
          pltpu.VMEM((2 * _CG, 32), jnp.float32),
          pltpu.SemaphoreType.DMA((2,)),
      ],
  )
  def body(coords_hbm, tab0, tab1, tab2, tab3, out_hbm,
           cbuf, idxb, wbuf, rows, outb, sems):
    tables = (tab0, tab1, tab2, tab3)
    wid = lax.axis_index("s") * _NC + lax.axis_index("c")
    iota = lax.iota(jnp.int32, 16)

    def gather_refs(h, l, c):
      i = l * 4 + c
      return (tables[l].at[idxb.at[16 * h + i]],
              rows.at[pl.ds((16 * h + i) * _CG, _CG)])

    def prep(k, h):
      base = wid * pw + k * _CG
      pltpu.sync_copy(coords_hbm.at[:, pl.ds(base, _CG)],
                      cbuf.at[pl.ds(4 * h, 3)])

      @plsc.parallel_loop(0, _GROUPS, unroll=2)
      def idx_g(g):
        sl = pl.ds(g * 16, 16)
        xv = cbuf[4 * h + 0, sl]
        yv = cbuf[4 * h + 1, sl]
        zv = cbuf[4 * h + 2, sl]
        for l, res in enumerate(_RES):
          rm1 = float(res - 1)
          gx = jnp.clip(xv * rm1, 0.0, rm1)
          gy = jnp.clip(yv * rm1, 0.0, rm1)
          gz = jnp.clip(zv * rm1, 0.0, rm1)
          fx = gx.astype(jnp.int32)
          fy = gy.astype(jnp.int32)
          fz = gz.astype(jnp.int32)
          wbuf[12 * h + l * 3 + 0, sl] = gx - fx.astype(jnp.float32)
          wbuf[12 * h + l * 3 + 1, sl] = gy - fy.astype(jnp.float32)
          wbuf[12 * h + l * 3 + 2, sl] = gz - fz.astype(jnp.float32)
          x1 = jnp.minimum(fx + 1, res - 1)
          y1 = jnp.minimum(fy + 1, res - 1)
          idxb[16 * h + l * 4 + 0, sl] = (fx * res + fy) * res + fz
          idxb[16 * h + l * 4 + 1, sl] = (fx * res + y1) * res + fz
          idxb[16 * h + l * 4 + 2, sl] = (x1 * res + fy) * res + fz
          idxb[16 * h + l * 4 + 3, sl] = (x1 * res + y1) * res + fz

      for l in range(4):
        for c in range(4):
          src, dst = gather_refs(h, l, c)
          pltpu.async_copy(src, dst, sems.at[h])

    def drain(h):
      pltpu.make_async_copy(tab3.at[pl.ds(0, 16 * _CG)],
                            rows.at[pl.ds(16 * h * _CG, 16 * _CG)],
                            sems.at[h]).wait()

    def interp(k, h):
      base = wid * pw + k * _CG

      @plsc.parallel_loop(0, _GROUPS, unroll=2)
      def interp_g(g):
        rb = iota + g * 16
        sl = pl.ds(g * 16, 16)
        for l in range(4):
          wx = wbuf[12 * h + l * 3 + 0, sl]
          wy = wbuf[12 * h + l * 3 + 1, sl]
          wz = wbuf[12 * h + l * 3 + 2, sl]
          omx = 1.0 - wx
          omy = 1.0 - wy
          p0 = omx * omy
          p1 = omx * wy
          p2 = wx * omy
          p3 = wx * wy
          rbs = [rb + (16 * h + l * 4 + c) * _CG for c in range(4)]
          orow = rb + h * _CG
          for f in range(8):
            col0 = jnp.full((16,), f, jnp.int32)
            col1 = jnp.full((16,), 8 + f, jnp.int32)
            a0 = plsc.load_gather(rows, [rbs[0], col0])
            b0 = plsc.load_gather(rows, [rbs[0], col1])
            a1 = plsc.load_gather(rows, [rbs[1], col0])
            b1 = plsc.load_gather(rows, [rbs[1], col1])
            a2 = plsc.load_gather(rows, [rbs[2], col0])
            b2 = plsc.load_gather(rows, [rbs[2], col1])
            a3 = plsc.load_gather(rows, [rbs[3], col0])
            b3 = plsc.load_gather(rows, [rbs[3], col1])
            s0 = (a0 * p0 + a1 * p1) + (a2 * p2 + a3 * p3)
            s1 = (b0 * p0 + b1 * p1) + (b2 * p2 + b3 * p3)
            o = s0 + (s1 - s0) * wz
            plsc.store_scatter(
                outb, [orow, jnp.full((16,), l * 8 + f, jnp.int32)], o)

      pltpu.sync_copy(outb.at[pl.ds(h * _CG, _CG)],
                      out_hbm.at[pl.ds(base, _CG)])

    prep(0, 0)

    def chunk_body(k, carry):
      h = lax.rem(k, 2)
      hn = lax.rem(k + 1, 2)

      @pl.when(k + 1 < nchunk)
      def _():
        prep(k + 1, hn)

      drain(h)
      interp(k, h)
      return carry

    lax.fori_loop(0, nchunk, chunk_body, 0)

  return body(coords_t, t0, t1, t2, t3)


def _mlp_body(x_ref, w0, b0, w1, b1, w2, b2, w3, b3, o_ref):
  x = x_ref[...]
  h = jnp.dot(x, w0[...], preferred_element_type=jnp.float32) + b0[...]
  h = jnp.maximum(h, 0.0)
  h = jnp.dot(h, w1[...], preferred_element_type=jnp.float32) + b1[...]
  h = jnp.maximum(h, 0.0)
  h = jnp.dot(h, w2[...], preferred_element_type=jnp.float32) + b2[...]
  h = jnp.maximum(h, 0.0)
  o_ref[...] = jnp.dot(h, w3[...], preferred_element_type=jnp.float32) + b3[...]


def _tc_mlp(feats, W0, b0, W1, b1, W2, b2, W3, b3):
  n = feats.shape[0]
  bn = 4096
  full = lambda shape: pl.BlockSpec(shape, lambda i: (0,) * len(shape))
  return pl.pallas_call(
      _mlp_body,
      grid=(n // bn,),
      in_specs=[
          pl.BlockSpec((bn, 32), lambda i: (i, 0)),
          full(W0.shape), full((1, 64)),
          full(W1.shape), full((1, 64)),
          full(W2.shape), full((1, 64)),
          full(W3.shape), full((1, 4)),
      ],
      out_specs=pl.BlockSpec((bn, 4), lambda i: (i, 0)),
      out_shape=jax.ShapeDtypeStruct((n, 4), jnp.float32),
  )(feats, W0, b0.reshape(1, 64), W1, b1.reshape(1, 64),
    W2, b2.reshape(1, 64), W3, b3.reshape(1, 4))


def _pair_z(g):
  res = g.shape[0]
  gz1 = jnp.concatenate([g[:, :, 1:, :], g[:, :, -1:, :]], axis=2)
  return jnp.concatenate([g, gz1], axis=-1).reshape(res * res * res, 16)


def kernel(coords, grid0, grid1, grid2, grid3, W0, b0, W1, b1, W2, b2, W3, b3):
  n = coords.shape[0]
  coords_t = coords.T
  t0 = _pair_z(grid0)
  t1 = _pair_z(grid1)
  t2 = _pair_z(grid2)
  t3 = _pair_z(grid3)
  feats = _sc_interp(coords_t, t0, t1, t2, t3, n)
  return _tc_mlp(feats, W0, b0, W1, b1, W2, b2, W3, b3)
